# Initial kernel scaffold; baseline (speedup 1.0000x reference)
import jax
import jax.numpy as jnp
from jax.experimental import pallas as pl


def kernel(dense_feat, center, offset, opacity, scale, rotation, feat_dc, keep_score, instance_affinity, motion_code, dynamic_logit, global_track_id):
    raise NotImplementedError("write your pallas kernel here")


# speedup vs baseline: 5.2876x; 5.2876x over previous
import functools

import jax
import jax.numpy as jnp
import numpy as np
from jax import lax
from jax.experimental import pallas as pl
from jax.experimental.pallas import tpu as pltpu
from jax.experimental.pallas import tpu_sc as plsc

B, T, V, H, W = 1, 2, 6, 64, 96
BT = B * T
N = V * H * W
NE = BT * N
NT = 8192
NSEG = BT * NT
THR2 = 4.0
DUP = 0.05
NEG = -1e30
BIG = 2**30

NC, NS = 2, 16
NWORK = NC * NS
SLICE = NSEG // NS

EPT1 = NE // NS
CH1 = 384
NCH1 = EPT1 // CH1

EPT = NE // NWORK
CH2 = 128
NCH2 = EPT // CH2
CH4 = 256
NCH4 = EPT // CH4

OPA_BLK, OPA_LANE = 2, 6
ROT_BLK, ROT_LO, ROT_HI = 2, 10, 14
KEEP_BLK, KEEP_LANE = 3, 1

_GDN = lax.GatherDimensionNumbers(
    offset_dims=(), collapsed_slice_dims=(0,), start_index_map=(0,))


def _iota():
  return lax.iota(jnp.int32, 16)


def _vg(x, idx):
  return lax.gather(x, idx[:, None], _GDN, (1,),
                    mode=lax.GatherScatterMode.PROMISE_IN_BOUNDS)


def _splat(vec16, lane):
  return _vg(vec16, jnp.broadcast_to(lane, (16,)).astype(jnp.int32))


def _seg_reduce(sseg, vals, op, ident):
  ii = _iota()
  cur = vals
  for d in (1, 2, 4, 8):
    sh = jnp.maximum(ii - d, 0)
    sv = _vg(cur, sh)
    ss = _vg(sseg, sh)
    ok = (ss == sseg) & (ii >= d)
    cur = op(cur, jnp.where(ok, sv, ident))
  nxt = _vg(sseg, jnp.minimum(ii + 1, 15))
  is_end = (sseg != nxt) | (ii == 15)
  return cur, is_end


def _rsqrt(x):
  i = plsc.bitcast(x, jnp.int32)
  y = plsc.bitcast(np.int32(0x5F3759DF) - (i >> 1), jnp.float32)
  for _ in range(3):
    y = y * (1.5 - 0.5 * x * y * y)
  return y


_MESH = plsc.VectorSubcoreMesh(core_axis_name="c", subcore_axis_name="s")
_CPARAMS = pltpu.CompilerParams(needs_layout_passes=False,
                                use_tc_tiling_on_sc=False)
_f32 = jnp.float32
_i32 = jnp.int32


def _sds(shape, dt):
  return jax.ShapeDtypeStruct(shape, dt)


@functools.partial(
    pl.kernel,
    out_type=(_sds((NE,), _f32), _sds((NE,), _f32), _sds((NSEG,), _f32),
              _sds((NSEG,), _f32), _sds((NSEG,), _i32)),
    mesh=_MESH,
    compiler_params=_CPARAMS,
    scratch_types=dict(
        idx2d=pltpu.VMEM((3, 128), _i32),
        bseg=pltpu.VMEM((CH1,), _i32),
        bkeep=pltpu.VMEM((CH1,), _f32),
        bcx=pltpu.VMEM((CH1,), _f32),
        bcy=pltpu.VMEM((CH1,), _f32),
        bcz=pltpu.VMEM((CH1,), _f32),
        be1=pltpu.VMEM((CH1,), _f32),
        bpx=pltpu.VMEM((CH1,), _f32),
        bpy=pltpu.VMEM((CH1,), _f32),
        bpz=pltpu.VMEM((CH1,), _f32),
        bones=pltpu.VMEM((CH1,), _f32),
        bact=pltpu.VMEM((CH1,), _f32),
        actfull=pltpu.VMEM((EPT1,), _f32),
        be2=pltpu.VMEM((CH1,), _f32),
        gcnt=pltpu.VMEM((CH1,), _f32),
        gs1=pltpu.VMEM((CH1,), _f32),
        gsx=pltpu.VMEM((CH1,), _f32),
        gsy=pltpu.VMEM((CH1,), _f32),
        gsz=pltpu.VMEM((CH1,), _f32),
        gm2=pltpu.VMEM((CH1,), _f32),
        m2_tbl=pltpu.VMEM((NSEG,), _f32),
        repi_tbl=pltpu.VMEM((NSEG,), _i32),
        accf=pltpu.VMEM((SLICE,), _f32),
        tmpf=pltpu.VMEM((SLICE,), _f32),
        acci=pltpu.VMEM((SLICE,), _i32),
        tmpi=pltpu.VMEM((SLICE,), _i32),
        zero1k=pltpu.VMEM((SLICE,), _f32),
        t_cnt=pltpu.VMEM_SHARED((NSEG,), _f32),
        t_s1=pltpu.VMEM_SHARED((NSEG,), _f32),
        t_sx=pltpu.VMEM_SHARED((NSEG,), _f32),
        t_sy=pltpu.VMEM_SHARED((NSEG,), _f32),
        t_sz=pltpu.VMEM_SHARED((NSEG,), _f32),
        t_s2=pltpu.VMEM_SHARED((NSEG,), _f32),
        t_m2=pltpu.VMEM_SHARED((NSEG,), _f32),
        stagef=pltpu.VMEM_SHARED((NS, NSEG), _f32),
        stagei=pltpu.VMEM_SHARED((NS, NSEG), _i32),
    ),
)
def _launch1(seg_h, keep_h, cx_h, cy_h, cz_h,
             act_o, e2_o, m2_o, s2_o, repi_o,
             idx2d, bseg, bkeep, bcx, bcy, bcz, be1, bpx, bpy, bpz, bones,
             bact, actfull, be2, gcnt, gs1, gsx, gsy, gsz, gm2, m2_tbl, repi_tbl,
             accf, tmpf, acci, tmpi, zero1k,
             t_cnt, t_s1, t_sx, t_sy, t_sz, t_s2, t_m2, stagef, stagei):
  cid = lax.axis_index("c")
  sid = lax.axis_index("s")

  def fill(ref, n, val):
    def body(i, _):
      ref[pl.ds(i * 16, 16)] = jnp.full((16,), val, ref.dtype)
      return 0
    lax.fori_loop(0, n // 16, body, 0)

  fill(zero1k, SLICE, 0.0)
  fill(bones, CH1, 1.0)
  fill(m2_tbl, NSEG, NEG)
  fill(repi_tbl, NSEG, 2**30)
  sl = pl.ds(sid * SLICE, SLICE)
  for t in (t_cnt, t_s1, t_sx, t_sy, t_sz, t_s2):
    pltpu.sync_copy(zero1k, t.at[sl])
  plsc.subcore_barrier()

  def load_chunk(base):
    for j in range(3):
      pltpu.sync_copy(seg_h.at[pl.ds(base + j * 128, 128)], idx2d.at[j])
    pltpu.sync_copy(seg_h.at[pl.ds(base, CH1)], bseg)
    pltpu.sync_copy(keep_h.at[pl.ds(base, CH1)], bkeep)
    pltpu.sync_copy(cx_h.at[pl.ds(base, CH1)], bcx)
    pltpu.sync_copy(cy_h.at[pl.ds(base, CH1)], bcy)
    pltpu.sync_copy(cz_h.at[pl.ds(base, CH1)], bcz)

  def gather_stats():
    for j in range(3):
      d = pl.ds(j * 128, 128)
      idx = idx2d.at[j]
      pltpu.sync_copy(t_cnt.at[idx], gcnt.at[d])
      pltpu.sync_copy(t_s1.at[idx], gs1.at[d])
      pltpu.sync_copy(t_sx.at[idx], gsx.at[d])
      pltpu.sync_copy(t_sy.at[idx], gsy.at[d])
      pltpu.sync_copy(t_sz.at[idx], gsz.at[d])

  def active_grp(g):
    d = pl.ds(g * 16, 16)
    s1 = jnp.maximum(gs1[d], 1e-30)
    mx = gsx[d] / s1
    my = gsy[d] / s1
    mz = gsz[d] / s1
    dx = bcx[d] - mx
    dy = bcy[d] - my
    dz = bcz[d] - mz
    d2 = dx * dx + dy * dy + dz * dz
    return (d2 <= THR2) & (gcnt[d] >= 2.0)

  def p1(c, _):
    base = sid * EPT1 + c * CH1
    load_chunk(base)
    def grp(g, _):
      d = pl.ds(g * 16, 16)
      e1 = jnp.exp(bkeep[d])
      be1[d] = e1
      bpx[d] = bcx[d] * e1
      bpy[d] = bcy[d] * e1
      bpz[d] = bcz[d] * e1
      return 0
    lax.fori_loop(0, CH1 // 16, grp, 0)
    for j in range(3):
      d = pl.ds(j * 128, 128)
      idx = idx2d.at[j]
      pltpu.sync_copy(bones.at[d], t_cnt.at[idx], add=True)
      pltpu.sync_copy(be1.at[d], t_s1.at[idx], add=True)
      pltpu.sync_copy(bpx.at[d], t_sx.at[idx], add=True)
      pltpu.sync_copy(bpy.at[d], t_sy.at[idx], add=True)
      pltpu.sync_copy(bpz.at[d], t_sz.at[idx], add=True)
    return 0
  lax.fori_loop(0, NCH1, p1, 0)
  plsc.subcore_barrier()

  def p2(c, _):
    base = sid * EPT1 + c * CH1
    load_chunk(base)
    gather_stats()
    for g in range(CH1 // 16):
      d = pl.ds(g * 16, 16)
      act = active_grp(g)
      av = jnp.where(act, 1.0, 0.0)
      bact[d] = av
      actfull[pl.ds(c * CH1 + g * 16, 16)] = av
      km = jnp.where(act, bkeep[d], NEG)
      sseg, perm = plsc.sort_key_val(bseg[d], _iota())
      kp = _vg(km, perm)
      cur, is_end = _seg_reduce(sseg, kp, jnp.maximum, NEG)
      old = plsc.load_gather(m2_tbl, [sseg])
      plsc.store_scatter(m2_tbl, [sseg], jnp.maximum(old, cur), mask=is_end)
    @pl.when(cid == 0)
    def _():
      pltpu.sync_copy(bact, act_o.at[pl.ds(base, CH1)])
    return 0
  lax.fori_loop(0, NCH1, p2, 0)

  pltpu.sync_copy(m2_tbl, stagef.at[sid])
  plsc.subcore_barrier()
  pltpu.sync_copy(stagef.at[0, sl], accf)
  def mrg_f(t, _):
    pltpu.sync_copy(stagef.at[t, sl], tmpf)
    def v(i, _):
      d = pl.ds(i * 16, 16)
      accf[d] = jnp.maximum(accf[d], tmpf[d])
      return 0
    lax.fori_loop(0, SLICE // 16, v, 0)
    return 0
  lax.fori_loop(1, NS, mrg_f, 0)
  pltpu.sync_copy(accf, t_m2.at[sl])
  @pl.when(cid == 0)
  def _():
    pltpu.sync_copy(accf, m2_o.at[sl])
  plsc.subcore_barrier()

  def p3(c, _):
    base = sid * EPT1 + c * CH1
    load_chunk(base)
    for j in range(3):
      pltpu.sync_copy(t_m2.at[idx2d.at[j]], gm2.at[pl.ds(j * 128, 128)])
    for g in range(CH1 // 16):
      d = pl.ds(g * 16, 16)
      act = actfull[pl.ds(c * CH1 + g * 16, 16)] > 0.5
      k16 = bkeep[d]
      m2v = gm2[d]
      e2 = jnp.where(act, jnp.exp(k16 - m2v), 0.0)
      be2[d] = e2
      cand = act & (k16 == m2v)
      eid = base + g * 16 + _iota()
      ci = jnp.where(cand, eid, BIG)
      sseg, perm = plsc.sort_key_val(bseg[d], _iota())
      cp = _vg(ci, perm)
      cur, is_end = _seg_reduce(sseg, cp, jnp.minimum, BIG)
      old = plsc.load_gather(repi_tbl, [sseg])
      plsc.store_scatter(repi_tbl, [sseg], jnp.minimum(old, cur), mask=is_end)
    @pl.when(cid == 0)
    def _():
      pltpu.sync_copy(be2, e2_o.at[pl.ds(base, CH1)])
    for j in range(3):
      pltpu.sync_copy(be2.at[pl.ds(j * 128, 128)], t_s2.at[idx2d.at[j]],
                      add=True)
    return 0
  lax.fori_loop(0, NCH1, p3, 0)

  pltpu.sync_copy(repi_tbl, stagei.at[sid])
  plsc.subcore_barrier()
  pltpu.sync_copy(stagei.at[0, sl], acci)
  def mrg_i(t, _):
    pltpu.sync_copy(stagei.at[t, sl], tmpi)
    def v(i, _):
      d = pl.ds(i * 16, 16)
      acci[d] = jnp.minimum(acci[d], tmpi[d])
      return 0
    lax.fori_loop(0, SLICE // 16, v, 0)
    return 0
  lax.fori_loop(1, NS, mrg_i, 0)
  @pl.when(cid == 0)
  def _():
    pltpu.sync_copy(acci, repi_o.at[sl])
    pltpu.sync_copy(t_s2.at[sl], s2_o.at[sl])


@functools.partial(
    pl.kernel,
    out_type=_sds((NC, 2, NSEG, 96), _f32),
    mesh=_MESH,
    compiler_params=_CPARAMS,
    scratch_types=dict(
        idx2d=pltpu.VMEM((1, 128), _i32),
        be2=pltpu.VMEM((CH2,), _f32),
        xbuf=pltpu.VMEM((CH2, 96), _f32),
        zbuf=pltpu.VMEM((64, 96), _f32),
        tbl=pltpu.VMEM_SHARED((NSEG, 96), _f32),
    ),
)
def _launch2(x0_h, x1_h, seg_h, e2_h, p_o, idx2d, be2, xbuf, zbuf, tbl):
  cid = lax.axis_index("c")
  sid = lax.axis_index("s")
  wid = cid * NS + sid

  def zb(i, _):
    def zr(k, _):
      zbuf[i, pl.ds(k * 16, 16)] = jnp.zeros((16,), _f32)
      return 0
    lax.fori_loop(0, 6, zr, 0)
    return 0
  lax.fori_loop(0, 64, zb, 0)

  for half in range(2):
    xh = x0_h if half == 0 else x1_h
    def zt(i, _):
      pltpu.sync_copy(zbuf, tbl.at[pl.ds(sid * SLICE + i * 64, 64), :])
      return 0
    lax.fori_loop(0, SLICE // 64, zt, 0)
    plsc.subcore_barrier()

    def chunk(c, _):
      base = wid * EPT + c * CH2
      pltpu.sync_copy(seg_h.at[pl.ds(base, 128)], idx2d.at[0])
      pltpu.sync_copy(e2_h.at[pl.ds(base, CH2)], be2)
      pltpu.sync_copy(xh.at[pl.ds(base, CH2), :], xbuf)
      def row(r, _):
        grp = (r // 16) * 16
        lane = r - grp
        ev = be2[pl.ds(grp, 16)]
        sp = _splat(ev, lane)
        for kb in range(6):
          d = pl.ds(kb * 16, 16)
          xbuf[r, d] = xbuf[r, d] * sp
        return 0
      lax.fori_loop(0, CH2, row, 0)
      pltpu.sync_copy(xbuf, tbl.at[idx2d.at[0]], add=True)
      return 0
    lax.fori_loop(0, NCH2, chunk, 0)
    plsc.subcore_barrier()
    def dump(i, _):
      d = pl.ds(sid * SLICE + i * 64, 64)
      pltpu.sync_copy(tbl.at[d, :], p_o.at[cid, half, d, :])
      return 0
    lax.fori_loop(0, SLICE // 64, dump, 0)
    plsc.subcore_barrier()


@functools.partial(
    pl.kernel,
    out_type=(_sds((NSEG, 96), _f32), _sds((NSEG, 96), _f32)),
    mesh=_MESH,
    compiler_params=_CPARAMS,
    scratch_types=dict(
        arow=pltpu.VMEM((128, 96), _f32),
        brow=pltpu.VMEM((128, 96), _f32),
        bs2=pltpu.VMEM((128,), _f32),
        bm2=pltpu.VMEM((128,), _f32),
    ),
)
def _launch3(p_h, s2_h, m2_h, m0_o, m1_o, arow, brow, bs2, bm2):
  cid = lax.axis_index("c")
  sid = lax.axis_index("s")
  wid = cid * NS + sid

  def chunk(c, _):
    rows = wid * (NSEG // NWORK) + c * 128
    pltpu.sync_copy(s2_h.at[pl.ds(rows, 128)], bs2)
    pltpu.sync_copy(m2_h.at[pl.ds(rows, 128)], bm2)
    for half in range(2):
      pltpu.sync_copy(p_h.at[0, half, pl.ds(rows, 128), :], arow)
      pltpu.sync_copy(p_h.at[1, half, pl.ds(rows, 128), :], brow)
      def row(r, _):
        grp = (r // 16) * 16
        lane = r - grp
        inv = 1.0 / jnp.maximum(_splat(bs2[pl.ds(grp, 16)], lane), 1e-30)
        for kb in range(6):
          d = pl.ds(kb * 16, 16)
          v = (arow[r, d] + brow[r, d]) * inv
          if half == 1 and kb == ROT_BLK:
            ii = _iota()
            rmask = (ii >= ROT_LO) & (ii < ROT_HI)
            sq = jnp.where(rmask, v * v, 0.0)
            u = sq + _vg(sq, jnp.minimum(ii + 1, 15))
            w = u + _vg(u, jnp.minimum(ii + 2, 15))
            nrm2 = _splat(w, ROT_LO)
            y = _rsqrt(jnp.maximum(nrm2, 1e-24))
            v = jnp.where(rmask, v * y, v)
          if half == 1 and kb == KEEP_BLK:
            m2sp = _splat(bm2[pl.ds(grp, 16)], lane)
            v = jnp.where(_iota() == KEEP_LANE, m2sp, v)
          arow[r, d] = v
        return 0
      lax.fori_loop(0, 128, row, 0)
      mo = m0_o if half == 0 else m1_o
      pltpu.sync_copy(arow, mo.at[pl.ds(rows, 128), :])
    return 0
  lax.fori_loop(0, NSEG // NWORK // 128, chunk, 0)


@functools.partial(
    pl.kernel,
    out_type=(_sds((NE, 96), _f32), _sds((NE, 96), _f32)),
    mesh=_MESH,
    compiler_params=_CPARAMS,
    scratch_types=dict(
        idx2d=pltpu.VMEM((2, 128), _i32),
        bseg=pltpu.VMEM((CH4,), _i32),
        bact=pltpu.VMEM((CH4,), _f32),
        bsc=pltpu.VMEM((CH4,), _f32),
        mbuf=pltpu.VMEM((CH4, 96), _f32),
        xbuf=pltpu.VMEM((CH4, 96), _f32),
        l_repi=pltpu.VMEM((NSEG,), _i32),
    ),
)
def _launch4(x0_h, x1_h, seg_h, act_h, repi_h, m0_h, m1_h, y0_o, y1_o,
             idx2d, bseg, bact, bsc, mbuf, xbuf, l_repi):
  cid = lax.axis_index("c")
  sid = lax.axis_index("s")
  wid = cid * NS + sid
  pltpu.sync_copy(repi_h, l_repi)

  def chunk(c, _):
    base = wid * EPT + c * CH4
    for j in range(2):
      pltpu.sync_copy(seg_h.at[pl.ds(base + j * 128, 128)], idx2d.at[j])
    pltpu.sync_copy(seg_h.at[pl.ds(base, CH4)], bseg)
    pltpu.sync_copy(act_h.at[pl.ds(base, CH4)], bact)
    for g in range(CH4 // 16):
      d = pl.ds(g * 16, 16)
      s16 = bseg[d]
      repi = plsc.load_gather(l_repi, [s16])
      eid = base + g * 16 + _iota()
      isrep = (bact[d] > 0.5) & (eid == repi)
      bsc[d] = jnp.where(isrep, 1.0, DUP)
    for half in range(2):
      mh = m0_h if half == 0 else m1_h
      xh = x0_h if half == 0 else x1_h
      yo = y0_o if half == 0 else y1_o
      for j in range(2):
        pltpu.sync_copy(mh.at[idx2d.at[j]], mbuf.at[pl.ds(j * 128, 128), :])
      pltpu.sync_copy(xh.at[pl.ds(base, CH4), :], xbuf)
      def row(r, _):
        grp = (r // 16) * 16
        lane = r - grp
        asp = _splat(bact[pl.ds(grp, 16)], lane) > 0.5
        scsp = _splat(bsc[pl.ds(grp, 16)], lane)
        for kb in range(6):
          d = pl.ds(kb * 16, 16)
          mv = mbuf[r, d]
          if half == 1 and kb == OPA_BLK:
            mv = jnp.where(_iota() == OPA_LANE, mv * scsp, mv)
          if half == 1 and kb == KEEP_BLK:
            mv = jnp.where(_iota() == KEEP_LANE, mv * scsp, mv)
          xbuf[r, d] = jnp.where(asp, mv, xbuf[r, d])
        return 0
      lax.fori_loop(0, CH4, row, 0)
      pltpu.sync_copy(xbuf, yo.at[pl.ds(base, CH4), :])
    return 0
  lax.fori_loop(0, NCH4, chunk, 0)


@jax.jit
def kernel(dense_feat, center, offset, opacity, scale, rotation, feat_dc,
           keep_score, instance_affinity, motion_code, dynamic_logit,
           global_track_id):
  n = NE
  ids = global_track_id.reshape(BT, N).astype(jnp.int32)
  seg = (ids + NT * jnp.arange(BT, dtype=jnp.int32)[:, None]).reshape(n)
  d = dense_feat.reshape(n, 128)
  c = center.reshape(n, 3)
  off = offset.reshape(n, 3)
  op = opacity.reshape(n, 1)
  sc = scale.reshape(n, 3)
  rot = rotation.reshape(n, 4)
  fdc = feat_dc.reshape(n, 3)
  kp = keep_score.reshape(n, 1)
  inst = instance_affinity.reshape(n, 16)
  mot = motion_code.reshape(n, 8)
  dyn = dynamic_logit.reshape(n, 1)
  x0 = d[:, :96]
  x1 = jnp.concatenate(
      [d[:, 96:], c, off, op, sc, rot, fdc, kp, inst, mot, dyn,
       jnp.zeros((n, 21), _f32)], axis=1)
  keep0 = kp[:, 0]

  act, e2, m2g, s2g, repig = _launch1(seg, keep0, c[:, 0], c[:, 1], c[:, 2])
  p = _launch2(x0, x1, seg, e2)
  m0, m1 = _launch3(p, s2g, m2g)
  y0, y1 = _launch4(x0, x1, seg, act, repig, m0, m1)
  return jnp.concatenate([y0, y1[:, :75]], axis=1).reshape(B, T, V, H, W, 171)
